# baseline (speedup 1.0000x reference)
import functools

import jax
import jax.numpy as jnp
from jax import lax
from jax.experimental import pallas as pl
from jax.experimental.pallas import tpu as pltpu
from jax.experimental.pallas import tpu_sc as plsc

N = 10000
E = 320000
D = 128
H = 128
C = 6

R = 1000
NG = N // R

NW = 32
BB = 64
NB = 160
IDXR = 80
EP = NW * NB * BB
NPAD = 10240
RPT = NPAD // 16
DUMMY = NPAD - 1


@functools.cache
def _build_sc_scatter():
    mesh = plsc.VectorSubcoreMesh(core_axis_name="c", subcore_axis_name="s")

    @functools.partial(
        pl.kernel,
        mesh=mesh,
        out_type=jax.ShapeDtypeStruct((2, NPAD, D), jnp.float32),
        scratch_types=[
            pltpu.VMEM((IDXR, 128), jnp.int32),
            pltpu.VMEM((8, BB), jnp.int32),
            pltpu.VMEM((4 * BB, D), jnp.float32),
            pltpu.VMEM_SHARED((NPAD, D), jnp.float32),
            pltpu.SemaphoreType.DMA((4,)),
            pltpu.SemaphoreType.DMA((4,)),
        ],
    )
    def sc_scatter(m_hbm, idx_hbm, zero_hbm, out_hbm,
                   idx_v, upk_v, rows_v, acc_s, gsems, ssems):
        cid = lax.axis_index("c")
        sid = lax.axis_index("s")
        wid = sid * 2 + cid
        pltpu.sync_copy(zero_hbm.at[pl.ds(sid * RPT, RPT)],
                        acc_s.at[pl.ds(sid * RPT, RPT)])
        pltpu.sync_copy(idx_hbm.at[wid], idx_v)
        plsc.subcore_barrier()

        rows = [rows_v.at[pl.ds(b * BB, BB)] for b in range(4)]

        def unpack_pair(r, base_slot):
            for k in range(8):
                v = idx_v[r, pl.ds(k * 16, 16)]
                sl = base_slot + (k // 4)
                lo = (k % 4) * 16
                upk_v[2 * sl, pl.ds(lo, 16)] = v & 0xFFFF
                upk_v[2 * sl + 1, pl.ds(lo, 16)] = (
                    lax.shift_right_logical(v, 16))

        def gather(b):
            pltpu.async_copy(m_hbm.at[upk_v.at[2 * b]], rows[b], gsems.at[b])

        def gwait(b):
            pltpu.make_async_copy(m_hbm.at[upk_v.at[2 * b]], rows[b],
                                  gsems.at[b]).wait()

        def scat(b):
            pltpu.async_copy(rows[b], acc_s.at[upk_v.at[2 * b + 1]],
                             ssems.at[b], add=True)

        def swait(b):
            pltpu.make_async_copy(rows[b], acc_s.at[upk_v.at[2 * b + 1]],
                                  ssems.at[b]).wait()

        unpack_pair(0, 0)
        gather(0)
        gather(1)
        unpack_pair(1, 2)
        gather(2)
        gather(3)

        def body(i, carry):
            for b in range(4):
                gwait(b)
                scat(b)
            swait(0)
            swait(1)
            unpack_pair(2 * i + 2, 0)
            gather(0)
            gather(1)
            swait(2)
            swait(3)
            unpack_pair(2 * i + 3, 2)
            gather(2)
            gather(3)
            return carry

        lax.fori_loop(0, NB // 4 - 1, body, 0)
        for b in range(4):
            gwait(b)
            scat(b)
        for b in range(4):
            swait(b)
        plsc.subcore_barrier()
        pltpu.sync_copy(acc_s.at[pl.ds(sid * RPT, RPT)],
                        out_hbm.at[cid, pl.ds(sid * RPT, RPT)])

    return sc_scatter


def _sc_scatter(m, idx, zeros_nd):
    return _build_sc_scatter()(m, idx, zeros_nd)


def _tcm1_body(x_ref, w1_ref, b1_ref, wg1_ref, m1_ref):
    h0 = jnp.dot(x_ref[...], w1_ref[...],
                 preferred_element_type=jnp.float32) + b1_ref[...]
    m1_ref[...] = jnp.dot(h0, wg1_ref[...], preferred_element_type=jnp.float32)


def _tcgh1_body(x_ref, w1_ref, b1_ref, whh_t, bhh, h0_ref, gh_ref):
    h0 = jnp.dot(x_ref[...], w1_ref[...],
                 preferred_element_type=jnp.float32) + b1_ref[...]
    h0_ref[...] = h0
    gh_ref[...] = jnp.dot(h0, whh_t[...],
                          preferred_element_type=jnp.float32) + bhh[...]


def _tcgh2_body(h_ref, whh_t, bhh, gh_ref):
    gh_ref[...] = jnp.dot(h_ref[...], whh_t[...],
                          preferred_element_type=jnp.float32) + bhh[...]


def _gru(p_ref, h, gh_ref, wih_t, bih):
    agg = p_ref[0] + p_ref[1]
    gi = jnp.dot(agg, wih_t[...], preferred_element_type=jnp.float32) + bih[...]
    gh = gh_ref[...]
    r = jax.nn.sigmoid(gi[:, :H] + gh[:, :H])
    z = jax.nn.sigmoid(gi[:, H:2 * H] + gh[:, H:2 * H])
    n = jnp.tanh(gi[:, 2 * H:] + r * gh[:, 2 * H:])
    return (1.0 - z) * n + z * h


def _tcgru1_body(p_ref, h0_ref, gh_ref, wih_t, bih, wg2_ref,
                 h1e_ref, m2_ref):
    h1 = _gru(p_ref, h0_ref[...], gh_ref, wih_t, bih)
    h1e = jnp.where(h1 > 0, h1, jnp.exp(jnp.minimum(h1, 0.0)) - 1.0)
    h1e_ref[...] = h1e
    m2_ref[...] = jnp.dot(h1e, wg2_ref[...], preferred_element_type=jnp.float32)


def _tcfinal_body(p_ref, h1e_ref, gh_ref, wih_t, bih, wc_ref, bc_ref, y_ref,
                  feat_ref, out_ref, loss_ref):
    feat = _gru(p_ref, h1e_ref[...], gh_ref, wih_t, bih)
    feat_ref[...] = feat
    outp = jnp.dot(feat, wc_ref[...],
                   preferred_element_type=jnp.float32) + bc_ref[...]
    out_ref[...] = outp
    col = lax.broadcasted_iota(jnp.int32, (R, D), 1)
    valid = col < C
    masked = jnp.where(valid, outp, -jnp.inf)
    mx = jnp.max(masked, axis=1, keepdims=True)
    s = jnp.sum(jnp.where(valid, jnp.exp(masked - mx), 0.0),
                axis=1, keepdims=True)
    lse = jnp.log(s) + mx
    y_blk = y_ref[0, 0, :]
    picked = jnp.sum(jnp.where(col == y_blk[:, None], outp, 0.0),
                     axis=1, keepdims=True)
    part = jnp.sum(lse - picked).reshape(1, 1)
    i = pl.program_id(0)
    prev = jnp.where(i == 0, jnp.zeros((1, 1), jnp.float32), loss_ref[...])
    tot = prev + part
    loss_ref[...] = jnp.where(i == NG - 1, tot / N, tot)


def _row_spec(shape):
    return pl.BlockSpec(shape, lambda i: (i,) + (0,) * (len(shape) - 1))


def _full_spec(shape):
    return pl.BlockSpec(shape, lambda i: (0,) * len(shape))


def kernel(x, edge_index, edge_attr, y, W1, b1, Wg1, Wih1, Whh1, bih1, bhh1,
           Wg2, Wih2, Whh2, bih2, bhh2, Wc, bc):
    f32 = jnp.float32
    ept = E // NW
    pad = NB * BB - ept
    src_pad = (jnp.arange(pad, dtype=jnp.int32)[None, :] * 37 +
               977 * jnp.arange(NW, dtype=jnp.int32)[:, None]) % N
    src_p = jnp.concatenate(
        [edge_index[0].astype(jnp.int32).reshape(NW, ept), src_pad], axis=1)
    spare = NPAD - N
    dst_pad = N + (jnp.arange(pad, dtype=jnp.int32)[None, :] +
                   (spare // NW) * jnp.arange(NW, dtype=jnp.int32)[:, None]
                   ) % spare
    dst_p = jnp.concatenate(
        [edge_index[1].astype(jnp.int32).reshape(NW, ept), dst_pad], axis=1)
    idx = (src_p | (dst_p << 16)).reshape(NW, IDXR, 128)
    zeros_nd = jnp.zeros((NPAD, D), f32)

    b1r = b1.reshape(1, H)
    bih1r = bih1.reshape(1, 3 * H)
    bhh1r = bhh1.reshape(1, 3 * H)
    bih2r = bih2.reshape(1, 3 * H)
    bhh2r = bhh2.reshape(1, 3 * H)
    wih1_t = Wih1.T
    whh1_t = Whh1.T
    wih2_t = Wih2.T
    whh2_t = Whh2.T
    wc_pad = jnp.pad(Wc, ((0, 0), (0, D - C)))
    bc_pad = jnp.pad(bc, (0, D - C)).reshape(1, D)
    y3 = y.astype(jnp.int32).reshape(NG, 1, R)

    m1 = pl.pallas_call(
        _tcm1_body,
        grid=(NG,),
        in_specs=[_row_spec((R, D)), _full_spec((D, H)), _full_spec((1, H)),
                  _full_spec((H, H))],
        out_specs=_row_spec((R, H)),
        out_shape=jax.ShapeDtypeStruct((N, H), f32),
    )(x, W1, b1r, Wg1)

    p1 = _sc_scatter(m1, idx, zeros_nd)

    h0, gh1 = pl.pallas_call(
        _tcgh1_body,
        grid=(NG,),
        in_specs=[_row_spec((R, D)), _full_spec((D, H)), _full_spec((1, H)),
                  _full_spec((H, 3 * H)), _full_spec((1, 3 * H))],
        out_specs=[_row_spec((R, H)), _row_spec((R, 3 * H))],
        out_shape=[jax.ShapeDtypeStruct((N, H), f32),
                   jax.ShapeDtypeStruct((N, 3 * H), f32)],
    )(x, W1, b1r, whh1_t, bhh1r)

    gru_in_specs = [
        pl.BlockSpec((2, R, H), lambda i: (0, i, 0)),
        _row_spec((R, H)),
        _row_spec((R, 3 * H)),
        _full_spec((H, 3 * H)),
        _full_spec((1, 3 * H)),
    ]

    h1e, m2 = pl.pallas_call(
        _tcgru1_body,
        grid=(NG,),
        in_specs=gru_in_specs + [_full_spec((H, H))],
        out_specs=[_row_spec((R, H)), _row_spec((R, H))],
        out_shape=[jax.ShapeDtypeStruct((N, H), f32),
                   jax.ShapeDtypeStruct((N, H), f32)],
    )(p1, h0, gh1, wih1_t, bih1r, Wg2)

    p2 = _sc_scatter(m2, idx, zeros_nd)

    gh2 = pl.pallas_call(
        _tcgh2_body,
        grid=(NG,),
        in_specs=[_row_spec((R, H)), _full_spec((H, 3 * H)),
                  _full_spec((1, 3 * H))],
        out_specs=_row_spec((R, 3 * H)),
        out_shape=jax.ShapeDtypeStruct((N, 3 * H), f32),
    )(h1e, whh2_t, bhh2r)

    feat, out_pad, loss_arr = pl.pallas_call(
        _tcfinal_body,
        grid=(NG,),
        in_specs=gru_in_specs + [_full_spec((H, D)), _full_spec((1, D)),
                                 pl.BlockSpec((1, 1, R), lambda i: (i, 0, 0))],
        out_specs=[_row_spec((R, H)), _row_spec((R, D)), _full_spec((1, 1))],
        out_shape=[jax.ShapeDtypeStruct((N, H), f32),
                   jax.ShapeDtypeStruct((N, D), f32),
                   jax.ShapeDtypeStruct((1, 1), f32)],
    )(p2, h1e, gh2, wih2_t, bih2r, wc_pad, bc_pad, y3)

    output = out_pad[:, :C]
    loss = loss_arr[0, 0]
    return (output, loss, feat)


# speedup vs baseline: 1.0927x; 1.0927x over previous
import functools

import jax
import jax.numpy as jnp
from jax import lax
from jax.experimental import pallas as pl
from jax.experimental.pallas import tpu as pltpu
from jax.experimental.pallas import tpu_sc as plsc

N = 10000
E = 320000
D = 128
H = 128
C = 6

R = 1000
NG = N // R

NW = 32
BB = 128
NB = 80
EP = NW * NB * BB
NPAD = 10240
RPT = NPAD // 16
DUMMY = NPAD - 1


@functools.cache
def _build_sc_scatter():
    mesh = plsc.VectorSubcoreMesh(core_axis_name="c", subcore_axis_name="s")

    @functools.partial(
        pl.kernel,
        mesh=mesh,
        out_type=jax.ShapeDtypeStruct((2, NPAD, D), jnp.float32),
        scratch_types=[
            pltpu.VMEM((NB, BB), jnp.int32),
            pltpu.VMEM((16, BB), jnp.int32),
            pltpu.VMEM((2 * BB, D), jnp.float32),
            pltpu.VMEM_SHARED((NPAD, D), jnp.float32),
            pltpu.SemaphoreType.DMA((2,)),
        ],
    )
    def sc_scatter(m_hbm, idx_hbm, zero_hbm, out_hbm,
                   idx_v, upk_v, rows_v, acc_s, sems):
        cid = lax.axis_index("c")
        sid = lax.axis_index("s")
        wid = sid * 2 + cid
        pltpu.sync_copy(zero_hbm.at[pl.ds(sid * RPT, RPT)],
                        acc_s.at[pl.ds(sid * RPT, RPT)])
        pltpu.sync_copy(idx_hbm.at[wid], idx_v)
        plsc.subcore_barrier()

        rows0 = rows_v.at[pl.ds(0, BB)]
        rows1 = rows_v.at[pl.ds(BB, BB)]
        sem0 = sems.at[0]
        sem1 = sems.at[1]

        def unpack(j, slot):
            for k in range(BB // 16):
                v = idx_v[j, pl.ds(k * 16, 16)]
                upk_v[2 * slot, pl.ds(k * 16, 16)] = v & 0xFFFF
                upk_v[2 * slot + 1, pl.ds(k * 16, 16)] = (
                    lax.shift_right_logical(v, 16))

        def gather(slot, rows, sem):
            pltpu.async_copy(m_hbm.at[upk_v.at[2 * slot]], rows, sem)

        def gwait(slot, rows, sem):
            pltpu.make_async_copy(m_hbm.at[upk_v.at[2 * slot]], rows,
                                  sem).wait()

        def scat(slot, rows):
            pltpu.sync_copy(rows, acc_s.at[upk_v.at[2 * slot + 1]], add=True)

        unpack(0, 0)
        gather(0, rows0, sem0)

        def body(i, carry):
            j0 = 2 * i
            unpack(j0 + 1, 1)
            gather(1, rows1, sem1)
            gwait(0, rows0, sem0)
            scat(0, rows0)
            unpack(j0 + 2, 0)
            gather(0, rows0, sem0)
            gwait(1, rows1, sem1)
            scat(1, rows1)
            return carry

        lax.fori_loop(0, NB // 2 - 1, body, 0)
        unpack(NB - 1, 1)
        gather(1, rows1, sem1)
        gwait(0, rows0, sem0)
        scat(0, rows0)
        gwait(1, rows1, sem1)
        scat(1, rows1)
        plsc.subcore_barrier()
        pltpu.sync_copy(acc_s.at[pl.ds(sid * RPT, RPT)],
                        out_hbm.at[cid, pl.ds(sid * RPT, RPT)])

    return sc_scatter


def _sc_scatter(m, idx, zeros_nd):
    return _build_sc_scatter()(m, idx, zeros_nd)


def _tc1_body(x_ref, w1_ref, b1_ref, wg1_ref, h0_ref, m1_ref):
    h0 = jnp.dot(x_ref[...], w1_ref[...],
                 preferred_element_type=jnp.float32) + b1_ref[...]
    h0_ref[...] = h0
    m1_ref[...] = jnp.dot(h0, wg1_ref[...], preferred_element_type=jnp.float32)


def _gru(p_ref, h_ref, wih_t, whh_t, bih, bhh):
    agg = p_ref[0] + p_ref[1]
    h = h_ref[...]
    gi = jnp.dot(agg, wih_t[...], preferred_element_type=jnp.float32) + bih[...]
    gh = jnp.dot(h, whh_t[...], preferred_element_type=jnp.float32) + bhh[...]
    r = jax.nn.sigmoid(gi[:, :H] + gh[:, :H])
    z = jax.nn.sigmoid(gi[:, H:2 * H] + gh[:, H:2 * H])
    n = jnp.tanh(gi[:, 2 * H:] + r * gh[:, 2 * H:])
    return (1.0 - z) * n + z * h


def _tc2_body(p_ref, h0_ref, wih_t, whh_t, bih, bhh, wg2_ref,
              h1e_ref, m2_ref):
    h1 = _gru(p_ref, h0_ref, wih_t, whh_t, bih, bhh)
    h1e = jnp.where(h1 > 0, h1, jnp.exp(jnp.minimum(h1, 0.0)) - 1.0)
    h1e_ref[...] = h1e
    m2_ref[...] = jnp.dot(h1e, wg2_ref[...], preferred_element_type=jnp.float32)


def _tc3_body(p_ref, h1e_ref, wih_t, whh_t, bih, bhh, wc_ref, bc_ref, y_ref,
              feat_ref, out_ref, loss_ref):
    feat = _gru(p_ref, h1e_ref, wih_t, whh_t, bih, bhh)
    feat_ref[...] = feat
    outp = jnp.dot(feat, wc_ref[...],
                   preferred_element_type=jnp.float32) + bc_ref[...]
    out_ref[...] = outp
    col = lax.broadcasted_iota(jnp.int32, (R, D), 1)
    valid = col < C
    masked = jnp.where(valid, outp, -jnp.inf)
    mx = jnp.max(masked, axis=1, keepdims=True)
    s = jnp.sum(jnp.where(valid, jnp.exp(masked - mx), 0.0),
                axis=1, keepdims=True)
    lse = jnp.log(s) + mx
    y_blk = y_ref[0, 0, :]
    picked = jnp.sum(jnp.where(col == y_blk[:, None], outp, 0.0),
                     axis=1, keepdims=True)
    part = jnp.sum(lse - picked).reshape(1, 1)
    i = pl.program_id(0)
    prev = jnp.where(i == 0, jnp.zeros((1, 1), jnp.float32), loss_ref[...])
    tot = prev + part
    loss_ref[...] = jnp.where(i == NG - 1, tot / N, tot)


def _row_spec(shape):
    return pl.BlockSpec(shape, lambda i: (i,) + (0,) * (len(shape) - 1))


def _full_spec(shape):
    return pl.BlockSpec(shape, lambda i: (0,) * len(shape))


def kernel(x, edge_index, edge_attr, y, W1, b1, Wg1, Wih1, Whh1, bih1, bhh1,
           Wg2, Wih2, Whh2, bih2, bhh2, Wc, bc):
    f32 = jnp.float32
    ept = E // NW
    pad = NB * BB - ept
    src_pad = (jnp.arange(pad, dtype=jnp.int32)[None, :] * 37 +
               977 * jnp.arange(NW, dtype=jnp.int32)[:, None]) % N
    src_p = jnp.concatenate(
        [edge_index[0].astype(jnp.int32).reshape(NW, ept), src_pad], axis=1)
    spare = NPAD - N
    dst_pad = N + (jnp.arange(pad, dtype=jnp.int32)[None, :] +
                   (spare // NW) * jnp.arange(NW, dtype=jnp.int32)[:, None]
                   ) % spare
    dst_p = jnp.concatenate(
        [edge_index[1].astype(jnp.int32).reshape(NW, ept), dst_pad], axis=1)
    idx = (src_p | (dst_p << 16)).reshape(NW, NB, BB)
    zeros_nd = jnp.zeros((NPAD, D), f32)

    b1r = b1.reshape(1, H)
    bih1r = bih1.reshape(1, 3 * H)
    bhh1r = bhh1.reshape(1, 3 * H)
    bih2r = bih2.reshape(1, 3 * H)
    bhh2r = bhh2.reshape(1, 3 * H)
    wih1_t = Wih1.T
    whh1_t = Whh1.T
    wih2_t = Wih2.T
    whh2_t = Whh2.T
    wc_pad = jnp.pad(Wc, ((0, 0), (0, D - C)))
    bc_pad = jnp.pad(bc, (0, D - C)).reshape(1, D)
    y3 = y.astype(jnp.int32).reshape(NG, 1, R)

    h0, m1 = pl.pallas_call(
        _tc1_body,
        grid=(NG,),
        in_specs=[_row_spec((R, D)), _full_spec((D, H)), _full_spec((1, H)),
                  _full_spec((H, H))],
        out_specs=[_row_spec((R, H)), _row_spec((R, H))],
        out_shape=[jax.ShapeDtypeStruct((N, H), f32),
                   jax.ShapeDtypeStruct((N, H), f32)],
    )(x, W1, b1r, Wg1)

    p1 = _sc_scatter(m1, idx, zeros_nd)

    gru_in_specs = [
        pl.BlockSpec((2, R, H), lambda i: (0, i, 0)),
        _row_spec((R, H)),
        _full_spec((H, 3 * H)), _full_spec((H, 3 * H)),
        _full_spec((1, 3 * H)), _full_spec((1, 3 * H)),
    ]

    h1e, m2 = pl.pallas_call(
        _tc2_body,
        grid=(NG,),
        in_specs=gru_in_specs + [_full_spec((H, H))],
        out_specs=[_row_spec((R, H)), _row_spec((R, H))],
        out_shape=[jax.ShapeDtypeStruct((N, H), f32),
                   jax.ShapeDtypeStruct((N, H), f32)],
    )(p1, h0, wih1_t, whh1_t, bih1r, bhh1r, Wg2)

    p2 = _sc_scatter(m2, idx, zeros_nd)

    feat, out_pad, loss_arr = pl.pallas_call(
        _tc3_body,
        grid=(NG,),
        in_specs=gru_in_specs + [_full_spec((H, D)), _full_spec((1, D)),
                                 pl.BlockSpec((1, 1, R), lambda i: (i, 0, 0))],
        out_specs=[_row_spec((R, H)), _row_spec((R, D)), _full_spec((1, 1))],
        out_shape=[jax.ShapeDtypeStruct((N, H), f32),
                   jax.ShapeDtypeStruct((N, D), f32),
                   jax.ShapeDtypeStruct((1, 1), f32)],
    )(p2, h1e, wih2_t, whh2_t, bih2r, bhh2r, wc_pad, bc_pad, y3)

    output = out_pad[:, :C]
    loss = loss_arr[0, 0]
    return (output, loss, feat)


# speedup vs baseline: 1.1278x; 1.0322x over previous
import functools

import jax
import jax.numpy as jnp
from jax import lax
from jax.experimental import pallas as pl
from jax.experimental.pallas import tpu as pltpu
from jax.experimental.pallas import tpu_sc as plsc

N = 10000
E = 320000
D = 128
H = 128
C = 6

R = 2000
NG = N // R

NW = 32
BB = 128
NB = 80
EP = NW * NB * BB
NPAD = 10240
RPT = NPAD // 16
DUMMY = NPAD - 1


@functools.cache
def _build_sc_scatter():
    mesh = plsc.VectorSubcoreMesh(core_axis_name="c", subcore_axis_name="s")

    @functools.partial(
        pl.kernel,
        mesh=mesh,
        out_type=jax.ShapeDtypeStruct((2, NPAD, D), jnp.float32),
        scratch_types=[
            pltpu.VMEM((NB, BB), jnp.int32),
            pltpu.VMEM((16, BB), jnp.int32),
            pltpu.VMEM((2 * BB, D), jnp.float32),
            pltpu.VMEM_SHARED((NPAD, D), jnp.float32),
            pltpu.SemaphoreType.DMA((2,)),
        ],
    )
    def sc_scatter(m_hbm, idx_hbm, zero_hbm, out_hbm,
                   idx_v, upk_v, rows_v, acc_s, sems):
        cid = lax.axis_index("c")
        sid = lax.axis_index("s")
        wid = sid * 2 + cid
        pltpu.sync_copy(zero_hbm.at[pl.ds(sid * RPT, RPT)],
                        acc_s.at[pl.ds(sid * RPT, RPT)])
        pltpu.sync_copy(idx_hbm.at[wid], idx_v)
        plsc.subcore_barrier()

        rows0 = rows_v.at[pl.ds(0, BB)]
        rows1 = rows_v.at[pl.ds(BB, BB)]
        sem0 = sems.at[0]
        sem1 = sems.at[1]

        def unpack(j, slot):
            for k in range(BB // 16):
                v = idx_v[j, pl.ds(k * 16, 16)]
                upk_v[2 * slot, pl.ds(k * 16, 16)] = v & 0xFFFF
                upk_v[2 * slot + 1, pl.ds(k * 16, 16)] = (
                    lax.shift_right_logical(v, 16))

        def gather(slot, rows, sem):
            pltpu.async_copy(m_hbm.at[upk_v.at[2 * slot]], rows, sem)

        def gwait(slot, rows, sem):
            pltpu.make_async_copy(m_hbm.at[upk_v.at[2 * slot]], rows,
                                  sem).wait()

        def scat(slot, rows):
            pltpu.sync_copy(rows, acc_s.at[upk_v.at[2 * slot + 1]], add=True)

        unpack(0, 0)
        gather(0, rows0, sem0)

        def body(i, carry):
            j0 = 2 * i
            unpack(j0 + 1, 1)
            gather(1, rows1, sem1)
            gwait(0, rows0, sem0)
            scat(0, rows0)
            unpack(j0 + 2, 0)
            gather(0, rows0, sem0)
            gwait(1, rows1, sem1)
            scat(1, rows1)
            return carry

        lax.fori_loop(0, NB // 2 - 1, body, 0)
        unpack(NB - 1, 1)
        gather(1, rows1, sem1)
        gwait(0, rows0, sem0)
        scat(0, rows0)
        gwait(1, rows1, sem1)
        scat(1, rows1)
        plsc.subcore_barrier()
        pltpu.sync_copy(acc_s.at[pl.ds(sid * RPT, RPT)],
                        out_hbm.at[cid, pl.ds(sid * RPT, RPT)])

    return sc_scatter


def _sc_scatter(m, idx, zeros_nd):
    return _build_sc_scatter()(m, idx, zeros_nd)


def _tc1_body(x_ref, w1_ref, b1_ref, wg1_ref, h0_ref, m1_ref):
    h0 = jnp.dot(x_ref[...], w1_ref[...],
                 preferred_element_type=jnp.float32) + b1_ref[...]
    h0_ref[...] = h0
    m1_ref[...] = jnp.dot(h0, wg1_ref[...], preferred_element_type=jnp.float32)


def _gru(p_ref, h_ref, wih_t, whh_t, bih, bhh):
    agg = p_ref[0] + p_ref[1]
    h = h_ref[...]
    gi = jnp.dot(agg, wih_t[...], preferred_element_type=jnp.float32) + bih[...]
    gh = jnp.dot(h, whh_t[...], preferred_element_type=jnp.float32) + bhh[...]
    r = jax.nn.sigmoid(gi[:, :H] + gh[:, :H])
    z = jax.nn.sigmoid(gi[:, H:2 * H] + gh[:, H:2 * H])
    n = jnp.tanh(gi[:, 2 * H:] + r * gh[:, 2 * H:])
    return (1.0 - z) * n + z * h


def _tc2_body(p_ref, h0_ref, wih_t, whh_t, bih, bhh, wg2_ref,
              h1e_ref, m2_ref):
    h1 = _gru(p_ref, h0_ref, wih_t, whh_t, bih, bhh)
    h1e = jnp.where(h1 > 0, h1, jnp.exp(jnp.minimum(h1, 0.0)) - 1.0)
    h1e_ref[...] = h1e
    m2_ref[...] = jnp.dot(h1e, wg2_ref[...], preferred_element_type=jnp.float32)


def _tc3_body(p_ref, h1e_ref, wih_t, whh_t, bih, bhh, wc_ref, bc_ref, y_ref,
              feat_ref, out_ref, loss_ref):
    feat = _gru(p_ref, h1e_ref, wih_t, whh_t, bih, bhh)
    feat_ref[...] = feat
    outp = jnp.dot(feat, wc_ref[...],
                   preferred_element_type=jnp.float32) + bc_ref[...]
    out_ref[...] = outp
    col = lax.broadcasted_iota(jnp.int32, (R, D), 1)
    valid = col < C
    masked = jnp.where(valid, outp, -jnp.inf)
    mx = jnp.max(masked, axis=1, keepdims=True)
    s = jnp.sum(jnp.where(valid, jnp.exp(masked - mx), 0.0),
                axis=1, keepdims=True)
    lse = jnp.log(s) + mx
    y_blk = y_ref[0, 0, :]
    picked = jnp.sum(jnp.where(col == y_blk[:, None], outp, 0.0),
                     axis=1, keepdims=True)
    part = jnp.sum(lse - picked).reshape(1, 1)
    i = pl.program_id(0)
    prev = jnp.where(i == 0, jnp.zeros((1, 1), jnp.float32), loss_ref[...])
    tot = prev + part
    loss_ref[...] = jnp.where(i == NG - 1, tot / N, tot)


def _row_spec(shape):
    return pl.BlockSpec(shape, lambda i: (i,) + (0,) * (len(shape) - 1))


def _full_spec(shape):
    return pl.BlockSpec(shape, lambda i: (0,) * len(shape))


def kernel(x, edge_index, edge_attr, y, W1, b1, Wg1, Wih1, Whh1, bih1, bhh1,
           Wg2, Wih2, Whh2, bih2, bhh2, Wc, bc):
    f32 = jnp.float32
    ept = E // NW
    pad = NB * BB - ept
    src_pad = (jnp.arange(pad, dtype=jnp.int32)[None, :] * 37 +
               977 * jnp.arange(NW, dtype=jnp.int32)[:, None]) % N
    src_p = jnp.concatenate(
        [edge_index[0].astype(jnp.int32).reshape(NW, ept), src_pad], axis=1)
    spare = NPAD - N
    dst_pad = N + (jnp.arange(pad, dtype=jnp.int32)[None, :] +
                   (spare // NW) * jnp.arange(NW, dtype=jnp.int32)[:, None]
                   ) % spare
    dst_p = jnp.concatenate(
        [edge_index[1].astype(jnp.int32).reshape(NW, ept), dst_pad], axis=1)
    idx = (src_p | (dst_p << 16)).reshape(NW, NB, BB)
    zeros_nd = jnp.zeros((NPAD, D), f32)

    b1r = b1.reshape(1, H)
    bih1r = bih1.reshape(1, 3 * H)
    bhh1r = bhh1.reshape(1, 3 * H)
    bih2r = bih2.reshape(1, 3 * H)
    bhh2r = bhh2.reshape(1, 3 * H)
    wih1_t = Wih1.T
    whh1_t = Whh1.T
    wih2_t = Wih2.T
    whh2_t = Whh2.T
    wc_pad = jnp.pad(Wc, ((0, 0), (0, D - C)))
    bc_pad = jnp.pad(bc, (0, D - C)).reshape(1, D)
    y3 = y.astype(jnp.int32).reshape(NG, 1, R)

    h0, m1 = pl.pallas_call(
        _tc1_body,
        grid=(NG,),
        in_specs=[_row_spec((R, D)), _full_spec((D, H)), _full_spec((1, H)),
                  _full_spec((H, H))],
        out_specs=[_row_spec((R, H)), _row_spec((R, H))],
        out_shape=[jax.ShapeDtypeStruct((N, H), f32),
                   jax.ShapeDtypeStruct((N, H), f32)],
    )(x, W1, b1r, Wg1)

    p1 = _sc_scatter(m1, idx, zeros_nd)

    gru_in_specs = [
        pl.BlockSpec((2, R, H), lambda i: (0, i, 0)),
        _row_spec((R, H)),
        _full_spec((H, 3 * H)), _full_spec((H, 3 * H)),
        _full_spec((1, 3 * H)), _full_spec((1, 3 * H)),
    ]

    h1e, m2 = pl.pallas_call(
        _tc2_body,
        grid=(NG,),
        in_specs=gru_in_specs + [_full_spec((H, H))],
        out_specs=[_row_spec((R, H)), _row_spec((R, H))],
        out_shape=[jax.ShapeDtypeStruct((N, H), f32),
                   jax.ShapeDtypeStruct((N, H), f32)],
    )(p1, h0, wih1_t, whh1_t, bih1r, bhh1r, Wg2)

    p2 = _sc_scatter(m2, idx, zeros_nd)

    feat, out_pad, loss_arr = pl.pallas_call(
        _tc3_body,
        grid=(NG,),
        in_specs=gru_in_specs + [_full_spec((H, D)), _full_spec((1, D)),
                                 pl.BlockSpec((1, 1, R), lambda i: (i, 0, 0))],
        out_specs=[_row_spec((R, H)), _row_spec((R, D)), _full_spec((1, 1))],
        out_shape=[jax.ShapeDtypeStruct((N, H), f32),
                   jax.ShapeDtypeStruct((N, D), f32),
                   jax.ShapeDtypeStruct((1, 1), f32)],
    )(p2, h1e, wih2_t, whh2_t, bih2r, bhh2r, wc_pad, bc_pad, y3)

    output = out_pad[:, :C]
    loss = loss_arr[0, 0]
    return (output, loss, feat)


# speedup vs baseline: 1.1397x; 1.0105x over previous
import functools

import jax
import jax.numpy as jnp
from jax import lax
from jax.experimental import pallas as pl
from jax.experimental.pallas import tpu as pltpu
from jax.experimental.pallas import tpu_sc as plsc

N = 10000
E = 320000
D = 128
H = 128
C = 6

R = 5000
NG = N // R

NW = 32
BB = 128
NB = 80
EP = NW * NB * BB
NPAD = 10240
RPT = NPAD // 16
DUMMY = NPAD - 1


@functools.cache
def _build_sc_scatter():
    mesh = plsc.VectorSubcoreMesh(core_axis_name="c", subcore_axis_name="s")

    @functools.partial(
        pl.kernel,
        mesh=mesh,
        out_type=jax.ShapeDtypeStruct((2, NPAD, D), jnp.float32),
        scratch_types=[
            pltpu.VMEM((NB, BB), jnp.int32),
            pltpu.VMEM((16, BB), jnp.int32),
            pltpu.VMEM((2 * BB, D), jnp.float32),
            pltpu.VMEM_SHARED((NPAD, D), jnp.float32),
            pltpu.SemaphoreType.DMA((2,)),
        ],
    )
    def sc_scatter(m_hbm, idx_hbm, zero_hbm, out_hbm,
                   idx_v, upk_v, rows_v, acc_s, sems):
        cid = lax.axis_index("c")
        sid = lax.axis_index("s")
        wid = sid * 2 + cid
        pltpu.sync_copy(zero_hbm.at[pl.ds(sid * RPT, RPT)],
                        acc_s.at[pl.ds(sid * RPT, RPT)])
        pltpu.sync_copy(idx_hbm.at[wid], idx_v)
        plsc.subcore_barrier()

        rows0 = rows_v.at[pl.ds(0, BB)]
        rows1 = rows_v.at[pl.ds(BB, BB)]
        sem0 = sems.at[0]
        sem1 = sems.at[1]

        def unpack(j, slot):
            for k in range(BB // 16):
                v = idx_v[j, pl.ds(k * 16, 16)]
                upk_v[2 * slot, pl.ds(k * 16, 16)] = v & 0xFFFF
                upk_v[2 * slot + 1, pl.ds(k * 16, 16)] = (
                    lax.shift_right_logical(v, 16))

        def gather(slot, rows, sem):
            pltpu.async_copy(m_hbm.at[upk_v.at[2 * slot]], rows, sem)

        def gwait(slot, rows, sem):
            pltpu.make_async_copy(m_hbm.at[upk_v.at[2 * slot]], rows,
                                  sem).wait()

        def scat(slot, rows):
            pltpu.sync_copy(rows, acc_s.at[upk_v.at[2 * slot + 1]], add=True)

        unpack(0, 0)
        gather(0, rows0, sem0)

        def body(i, carry):
            j0 = 2 * i
            unpack(j0 + 1, 1)
            gather(1, rows1, sem1)
            gwait(0, rows0, sem0)
            scat(0, rows0)
            unpack(j0 + 2, 0)
            gather(0, rows0, sem0)
            gwait(1, rows1, sem1)
            scat(1, rows1)
            return carry

        lax.fori_loop(0, NB // 2 - 1, body, 0)
        unpack(NB - 1, 1)
        gather(1, rows1, sem1)
        gwait(0, rows0, sem0)
        scat(0, rows0)
        gwait(1, rows1, sem1)
        scat(1, rows1)
        plsc.subcore_barrier()
        pltpu.sync_copy(acc_s.at[pl.ds(sid * RPT, RPT)],
                        out_hbm.at[cid, pl.ds(sid * RPT, RPT)])

    return sc_scatter


def _sc_scatter(m, idx, zeros_nd):
    return _build_sc_scatter()(m, idx, zeros_nd)


def _tc1_body(x_ref, w1_ref, b1_ref, wg1_ref, h0_ref, m1_ref):
    h0 = jnp.dot(x_ref[...], w1_ref[...],
                 preferred_element_type=jnp.float32) + b1_ref[...]
    h0_ref[...] = h0
    m1_ref[...] = jnp.dot(h0, wg1_ref[...], preferred_element_type=jnp.float32)


def _gru(p_ref, h_ref, wih_t, whh_t, bih, bhh):
    agg = p_ref[0] + p_ref[1]
    h = h_ref[...]
    gi = jnp.dot(agg, wih_t[...], preferred_element_type=jnp.float32) + bih[...]
    gh = jnp.dot(h, whh_t[...], preferred_element_type=jnp.float32) + bhh[...]
    r = jax.nn.sigmoid(gi[:, :H] + gh[:, :H])
    z = jax.nn.sigmoid(gi[:, H:2 * H] + gh[:, H:2 * H])
    n = jnp.tanh(gi[:, 2 * H:] + r * gh[:, 2 * H:])
    return (1.0 - z) * n + z * h


def _tc2_body(p_ref, h0_ref, wih_t, whh_t, bih, bhh, wg2_ref,
              h1e_ref, m2_ref):
    h1 = _gru(p_ref, h0_ref, wih_t, whh_t, bih, bhh)
    h1e = jnp.where(h1 > 0, h1, jnp.exp(jnp.minimum(h1, 0.0)) - 1.0)
    h1e_ref[...] = h1e
    m2_ref[...] = jnp.dot(h1e, wg2_ref[...], preferred_element_type=jnp.float32)


def _tc3_body(p_ref, h1e_ref, wih_t, whh_t, bih, bhh, wc_ref, bc_ref, y_ref,
              feat_ref, out_ref, loss_ref):
    feat = _gru(p_ref, h1e_ref, wih_t, whh_t, bih, bhh)
    feat_ref[...] = feat
    outp = jnp.dot(feat, wc_ref[...],
                   preferred_element_type=jnp.float32) + bc_ref[...]
    out_ref[...] = outp
    col = lax.broadcasted_iota(jnp.int32, (R, D), 1)
    valid = col < C
    masked = jnp.where(valid, outp, -jnp.inf)
    mx = jnp.max(masked, axis=1, keepdims=True)
    s = jnp.sum(jnp.where(valid, jnp.exp(masked - mx), 0.0),
                axis=1, keepdims=True)
    lse = jnp.log(s) + mx
    y_blk = y_ref[0, 0, :]
    picked = jnp.sum(jnp.where(col == y_blk[:, None], outp, 0.0),
                     axis=1, keepdims=True)
    part = jnp.sum(lse - picked).reshape(1, 1)
    i = pl.program_id(0)
    prev = jnp.where(i == 0, jnp.zeros((1, 1), jnp.float32), loss_ref[...])
    tot = prev + part
    loss_ref[...] = jnp.where(i == NG - 1, tot / N, tot)


def _row_spec(shape):
    return pl.BlockSpec(shape, lambda i: (i,) + (0,) * (len(shape) - 1))


def _full_spec(shape):
    return pl.BlockSpec(shape, lambda i: (0,) * len(shape))


def kernel(x, edge_index, edge_attr, y, W1, b1, Wg1, Wih1, Whh1, bih1, bhh1,
           Wg2, Wih2, Whh2, bih2, bhh2, Wc, bc):
    f32 = jnp.float32
    ept = E // NW
    pad = NB * BB - ept
    src_pad = (jnp.arange(pad, dtype=jnp.int32)[None, :] * 37 +
               977 * jnp.arange(NW, dtype=jnp.int32)[:, None]) % N
    src_p = jnp.concatenate(
        [edge_index[0].astype(jnp.int32).reshape(NW, ept), src_pad], axis=1)
    spare = NPAD - N
    dst_pad = N + (jnp.arange(pad, dtype=jnp.int32)[None, :] +
                   (spare // NW) * jnp.arange(NW, dtype=jnp.int32)[:, None]
                   ) % spare
    dst_p = jnp.concatenate(
        [edge_index[1].astype(jnp.int32).reshape(NW, ept), dst_pad], axis=1)
    idx = (src_p | (dst_p << 16)).reshape(NW, NB, BB)
    zeros_nd = jnp.zeros((NPAD, D), f32)

    b1r = b1.reshape(1, H)
    bih1r = bih1.reshape(1, 3 * H)
    bhh1r = bhh1.reshape(1, 3 * H)
    bih2r = bih2.reshape(1, 3 * H)
    bhh2r = bhh2.reshape(1, 3 * H)
    wih1_t = Wih1.T
    whh1_t = Whh1.T
    wih2_t = Wih2.T
    whh2_t = Whh2.T
    wc_pad = jnp.pad(Wc, ((0, 0), (0, D - C)))
    bc_pad = jnp.pad(bc, (0, D - C)).reshape(1, D)
    y3 = y.astype(jnp.int32).reshape(NG, 1, R)

    h0, m1 = pl.pallas_call(
        _tc1_body,
        grid=(NG,),
        in_specs=[_row_spec((R, D)), _full_spec((D, H)), _full_spec((1, H)),
                  _full_spec((H, H))],
        out_specs=[_row_spec((R, H)), _row_spec((R, H))],
        out_shape=[jax.ShapeDtypeStruct((N, H), f32),
                   jax.ShapeDtypeStruct((N, H), f32)],
    )(x, W1, b1r, Wg1)

    p1 = _sc_scatter(m1, idx, zeros_nd)

    gru_in_specs = [
        pl.BlockSpec((2, R, H), lambda i: (0, i, 0)),
        _row_spec((R, H)),
        _full_spec((H, 3 * H)), _full_spec((H, 3 * H)),
        _full_spec((1, 3 * H)), _full_spec((1, 3 * H)),
    ]

    h1e, m2 = pl.pallas_call(
        _tc2_body,
        grid=(NG,),
        in_specs=gru_in_specs + [_full_spec((H, H))],
        out_specs=[_row_spec((R, H)), _row_spec((R, H))],
        out_shape=[jax.ShapeDtypeStruct((N, H), f32),
                   jax.ShapeDtypeStruct((N, H), f32)],
    )(p1, h0, wih1_t, whh1_t, bih1r, bhh1r, Wg2)

    p2 = _sc_scatter(m2, idx, zeros_nd)

    feat, out_pad, loss_arr = pl.pallas_call(
        _tc3_body,
        grid=(NG,),
        in_specs=gru_in_specs + [_full_spec((H, D)), _full_spec((1, D)),
                                 pl.BlockSpec((1, 1, R), lambda i: (i, 0, 0))],
        out_specs=[_row_spec((R, H)), _row_spec((R, D)), _full_spec((1, 1))],
        out_shape=[jax.ShapeDtypeStruct((N, H), f32),
                   jax.ShapeDtypeStruct((N, D), f32),
                   jax.ShapeDtypeStruct((1, 1), f32)],
    )(p2, h1e, wih2_t, whh2_t, bih2r, bhh2r, wc_pad, bc_pad, y3)

    output = out_pad[:, :C]
    loss = loss_arr[0, 0]
    return (output, loss, feat)
